# Initial kernel scaffold; baseline (speedup 1.0000x reference)
import jax
import jax.numpy as jnp
from jax.experimental import pallas as pl


def kernel(category, noise, edge_index, W1, b1, W2, b2):
    raise NotImplementedError("write your pallas kernel here")


# speedup vs baseline: 19.5309x; 19.5309x over previous
import functools

import jax
import jax.numpy as jnp
from jax import lax
from jax.experimental import pallas as pl
from jax.experimental.pallas import tpu as pltpu
from jax.experimental.pallas import tpu_sc as plsc

N = 100000
E = 1600000
NC = 2
NS = 16
NW = NC * NS
D = 16
CH = 1000
NCHUNK = E // NS // CH
WCH = 250
NPT = N // NS
DCH = 2000
DNCHUNK = E // NW // DCH

_f32 = jnp.float32


def _sc_mesh():
    return plsc.VectorSubcoreMesh(
        core_axis_name="c", subcore_axis_name="s", num_cores=NC, num_subcores=NS
    )


def _deg_kernel(dst, out, local, didx):
    c = lax.axis_index("c")
    s = lax.axis_index("s")
    wid = c * NS + s

    def zero(j, _):
        local[pl.ds(j * 16, 16)] = jnp.zeros((16,), _f32)
        return ()

    lax.fori_loop(0, N // 16, zero, ())
    ones = jnp.ones((16,), _f32)

    def chunk(i, _):
        base = wid * (E // NW) + i * DCH
        pltpu.sync_copy(dst.at[pl.ds(base, DCH)], didx)

        def inner(j, _):
            idx = didx[pl.ds(j * 16, 16)]
            plsc.addupdate_scatter(local, [idx], ones)
            return ()

        lax.fori_loop(0, DCH // 16, inner, ())
        return ()

    lax.fori_loop(0, DNCHUNK, chunk, ())
    pltpu.sync_copy(local, out.at[wid])


def _degree_partials(dst):
    k = pl.kernel(
        _deg_kernel,
        out_type=jax.ShapeDtypeStruct((NW, N), _f32),
        mesh=_sc_mesh(),
        scratch_types=dict(
            local=pltpu.VMEM((N,), _f32),
            didx=pltpu.VMEM((DCH,), jnp.int32),
        ),
        compiler_params=pltpu.CompilerParams(use_tc_tiling_on_sc=False, needs_layout_passes=False),
    )
    return k(dst)


def _prop_body(mode, tbl, src, dst, scale, extra, out, accum, sidx, didx,
               rbuf, sbuf, zbuf, sem):
    c = lax.axis_index("c")
    s = lax.axis_index("s")

    def zb(j, _):
        zbuf[j] = jnp.zeros((16,), _f32)
        return ()

    lax.fori_loop(0, WCH, zb, ())
    for t in range(NPT // WCH):
        pltpu.sync_copy(zbuf, accum.at[pl.ds(s * NPT + t * WCH, WCH), :])
    plsc.subcore_barrier()

    def chunk(i, _):
        base = s * (E // NS) + i * CH
        pltpu.sync_copy(src.at[pl.ds(base, CH)], sidx)
        pltpu.sync_copy(dst.at[pl.ds(base, CH)], didx)
        pltpu.async_copy(tbl.at[c].at[sidx], rbuf, sem).wait()
        pltpu.sync_copy(rbuf, accum.at[didx], add=True)
        return ()

    lax.fori_loop(0, NCHUNK, chunk, ())
    plsc.subcore_barrier()

    for t in range(NPT // WCH):
        r0 = s * NPT + t * WCH
        pltpu.sync_copy(accum.at[pl.ds(r0, WCH), :], rbuf.at[pl.ds(0, WCH), :])
        pltpu.sync_copy(scale.at[pl.ds(r0, WCH), :], sbuf)
        if mode > 0:
            pltpu.sync_copy(extra.at[c].at[pl.ds(r0, WCH), :], zbuf)

        def wrow(j, _):
            v = rbuf[j] * sbuf[j]
            if mode == 1:
                v = v + zbuf[j]
            elif mode == 2:
                v = v + zbuf[j]
                v = jnp.where(v > 0, v, 0.25 * v)
            rbuf[j] = v
            return ()

        lax.fori_loop(0, WCH, wrow, ())
        pltpu.sync_copy(rbuf.at[pl.ds(0, WCH), :], out.at[c].at[pl.ds(r0, WCH), :])


def _propagate(mode, tbl, src, dst, scale, extra):
    body = functools.partial(_prop_body, mode)
    k = pl.kernel(
        body,
        out_type=jax.ShapeDtypeStruct((NC, N, D), _f32),
        mesh=_sc_mesh(),
        scratch_types=dict(
            accum=pltpu.VMEM_SHARED((N, D), _f32),
            sidx=pltpu.VMEM((CH,), jnp.int32),
            didx=pltpu.VMEM((CH,), jnp.int32),
            rbuf=pltpu.VMEM((CH, D), _f32),
            sbuf=pltpu.VMEM((WCH, D), _f32),
            zbuf=pltpu.VMEM((WCH, D), _f32),
            sem=pltpu.SemaphoreType.DMA,
        ),
        compiler_params=pltpu.CompilerParams(use_tc_tiling_on_sc=False, needs_layout_passes=False),
    )
    if extra is None:
        extra = jnp.zeros((NC, 8, D), _f32)
    return k(tbl, src, dst, scale, extra)


BM = 2048


def _pre_body(p_ref, cat_ref, noi_ref, u0_ref, d2e_ref, d1e_ref, sqe_ref):
    deg = jnp.sum(p_ref[...], axis=0)
    pos = deg > 0
    dinv = jnp.where(pos, lax.rsqrt(jnp.where(pos, deg, 1.0)), 0.0)
    dinv2 = jnp.where(pos, 1.0 / jnp.where(pos, deg, 1.0), 0.0)
    sqd = jnp.sqrt(deg)
    u0_ref[0] = dinv[:, None] * cat_ref[...]
    u0_ref[1] = dinv[:, None] * noi_ref[...]
    d2e_ref[...] = jnp.broadcast_to(dinv2[:, None], (BM, D))
    d1e_ref[...] = jnp.broadcast_to(dinv[:, None], (BM, D))
    sqe_ref[...] = jnp.broadcast_to(sqd[:, None], (BM, D))


def _tc_pre(partials, category, noise):
    grid = (N + BM - 1) // BM
    fb = jax.ShapeDtypeStruct((N, D), _f32)
    return pl.pallas_call(
        _pre_body,
        grid=(grid,),
        in_specs=[
            pl.BlockSpec((NW, BM), lambda i: (0, i)),
            pl.BlockSpec((BM, D), lambda i: (i, 0)),
            pl.BlockSpec((BM, D), lambda i: (i, 0)),
        ],
        out_specs=[
            pl.BlockSpec((NC, BM, D), lambda i: (0, i, 0)),
            pl.BlockSpec((BM, D), lambda i: (i, 0)),
            pl.BlockSpec((BM, D), lambda i: (i, 0)),
            pl.BlockSpec((BM, D), lambda i: (i, 0)),
        ],
        out_shape=[jax.ShapeDtypeStruct((NC, N, D), _f32), fb, fb, fb],
    )(partials, category, noise)


def _main_body(cat_ref, noi_ref, u1_ref, u2_ref, u3_ref, sqe_ref, d1e_ref,
               w1t_ref, b1_ref, w2r_ref, b2_ref,
               z0_ref, zt1_ref, zt2_ref, zt3_ref):
    s = sqe_ref[...]
    cat8 = jnp.concatenate(
        [cat_ref[...], noi_ref[...],
         s * u1_ref[0], s * u1_ref[1],
         s * u2_ref[0], s * u2_ref[1],
         s * u3_ref[0], s * u3_ref[1]], axis=1)
    y = jnp.dot(cat8, w1t_ref[...], preferred_element_type=_f32) + b1_ref[...]
    h = jnp.where(y > 0, y, 0.25 * y)
    w2r = w2r_ref[...]
    z0 = jnp.dot(h, w2r[:, :32], preferred_element_type=_f32) + b2_ref[...]
    hs = d1e_ref[:, :1] * h
    zs = jnp.dot(hs, w2r[:, 32:], preferred_element_type=_f32)
    z0_ref[0], z0_ref[1] = z0[:, :D], z0[:, D:]
    zt1_ref[0], zt1_ref[1] = zs[:, 0:D], zs[:, D:2 * D]
    zt2_ref[0], zt2_ref[1] = zs[:, 2 * D:3 * D], zs[:, 3 * D:4 * D]
    zt3_ref[0], zt3_ref[1] = zs[:, 4 * D:5 * D], zs[:, 5 * D:6 * D]


def _tc_main(category, noise, u1, u2, u3, sqe, d1e, W1, b1, W2, b2):
    grid = (N + BM - 1) // BM
    w1t = W1.T
    w2r = jnp.concatenate(
        [W2[:, 64 * j:64 * (j + 1)].T for j in range(4)], axis=1)
    fspec = pl.BlockSpec((BM, D), lambda i: (i, 0))
    uspec = pl.BlockSpec((NC, BM, D), lambda i: (0, i, 0))
    ut = jax.ShapeDtypeStruct((NC, N, D), _f32)
    return pl.pallas_call(
        _main_body,
        grid=(grid,),
        in_specs=[
            fspec, fspec, uspec, uspec, uspec, fspec, fspec,
            pl.BlockSpec((128, 64), lambda i: (0, 0)),
            pl.BlockSpec((1, 64), lambda i: (0, 0)),
            pl.BlockSpec((64, 128), lambda i: (0, 0)),
            pl.BlockSpec((1, 32), lambda i: (0, 0)),
        ],
        out_specs=[uspec, uspec, uspec, uspec],
        out_shape=[ut, ut, ut, ut],
    )(category, noise, u1, u2, u3, sqe, d1e,
      w1t, b1.reshape(1, 64), w2r, b2.reshape(1, 32))


def kernel(category, noise, edge_index, W1, b1, W2, b2):
    src = edge_index[0].astype(jnp.int32)
    dst = edge_index[1].astype(jnp.int32)

    partials = _degree_partials(dst)
    u0, d2e, d1e, sqe = _tc_pre(partials, category, noise)

    u1 = _propagate(0, u0, src, dst, d2e, None)
    u2 = _propagate(0, u1, src, dst, d2e, None)
    u3 = _propagate(0, u2, src, dst, d2e, None)

    z0, zt1, zt2, zt3 = _tc_main(category, noise, u1, u2, u3, sqe, d1e,
                                 W1, b1, W2, b2)

    w = _propagate(1, zt3, src, dst, d2e, zt2)
    w = _propagate(1, w, src, dst, d2e, zt1)
    o = _propagate(2, w, src, dst, d1e, z0)

    return jnp.concatenate([o[0], o[1]], axis=1)


# speedup vs baseline: 25.7440x; 1.3181x over previous
import functools

import jax
import jax.numpy as jnp
from jax import lax
from jax.experimental import pallas as pl
from jax.experimental.pallas import tpu as pltpu
from jax.experimental.pallas import tpu_sc as plsc

N = 100000
E = 1600000
NC = 2
NS = 16
NW = NC * NS
D = 16
CH = 800
NCHUNK = E // NS // CH
EPT = E // NS
WCH = 250
NPT = N // NS
DCH = 2000
DNCHUNK = E // NW // DCH

_f32 = jnp.float32


def _sc_mesh():
    return plsc.VectorSubcoreMesh(
        core_axis_name="c", subcore_axis_name="s", num_cores=NC, num_subcores=NS
    )


def _deg_kernel(dst, out, local, didx):
    c = lax.axis_index("c")
    s = lax.axis_index("s")
    wid = c * NS + s

    def zero(j, _):
        local[pl.ds(j * 16, 16)] = jnp.zeros((16,), _f32)
        return ()

    lax.fori_loop(0, N // 16, zero, ())
    ones = jnp.ones((16,), _f32)

    def chunk(i, _):
        base = wid * (E // NW) + i * DCH
        pltpu.sync_copy(dst.at[pl.ds(base, DCH)], didx)

        def inner(j, _):
            idx = didx[pl.ds(j * 16, 16)]
            plsc.addupdate_scatter(local, [idx], ones)
            return ()

        lax.fori_loop(0, DCH // 16, inner, ())
        return ()

    lax.fori_loop(0, DNCHUNK, chunk, ())
    pltpu.sync_copy(local, out.at[wid])


def _degree_partials(dst):
    k = pl.kernel(
        _deg_kernel,
        out_type=jax.ShapeDtypeStruct((NW, N), _f32),
        mesh=_sc_mesh(),
        scratch_types=dict(
            local=pltpu.VMEM((N,), _f32),
            didx=pltpu.VMEM((DCH,), jnp.int32),
        ),
        compiler_params=pltpu.CompilerParams(use_tc_tiling_on_sc=False, needs_layout_passes=False),
    )
    return k(dst)


def _prop_body(mode, tbl, eidx, scale, extra, out, accum,
               eb0, eb1, rb0, rb1, gs0, gs1, ss0, ss1, is0, is1):
    c = lax.axis_index("c")
    s = lax.axis_index("s")
    ebs = (eb0, eb1)
    rbs = (rb0, rb1)
    gss = (gs0, gs1)
    sss = (ss0, ss1)
    iss = (is0, is1)

    def zb(j, _):
        rb0[j] = jnp.zeros((16,), _f32)
        return ()

    lax.fori_loop(0, WCH, zb, ())
    for t in range(NPT // WCH):
        pltpu.sync_copy(rb0.at[pl.ds(0, WCH), :],
                        accum.at[pl.ds(s * NPT + t * WCH, WCH), :])
    plsc.subcore_barrier()

    def idx_start(i, p):
        base = s * EPT + i * CH
        return pltpu.async_copy(eidx.at[:, pl.ds(base, CH)], ebs[p], iss[p])

    def gather_start(p):
        return pltpu.async_copy(tbl.at[c].at[ebs[p].at[0]], rbs[p], gss[p])

    def gather_wait(p):
        pltpu.make_async_copy(tbl.at[c].at[ebs[p].at[0]], rbs[p], gss[p]).wait()

    def scat_start(p):
        return pltpu.async_copy(rbs[p], accum.at[ebs[p].at[1]], sss[p], add=True)

    def scat_wait(p):
        pltpu.make_async_copy(rbs[p], accum.at[ebs[p].at[1]], sss[p]).wait()

    idx_start(0, 0).wait()
    gather_start(0)
    idx_start(1, 1).wait()
    gather_wait(0)
    scat_start(0)
    gather_start(1)

    def chunk_body(i, p):
        scat_wait(p)
        idesc = idx_start(i, p)
        gather_wait(1 - p)
        scat_start(1 - p)
        idesc.wait()
        gather_start(p)
        return ()

    def pair(k, _):
        i0 = 2 + 2 * k
        chunk_body(i0, 0)
        chunk_body(i0 + 1, 1)
        return ()

    lax.fori_loop(0, (NCHUNK - 2) // 2, pair, ())
    if (NCHUNK - 2) % 2 == 1:
        chunk_body(NCHUNK - 1, 0)
        last = 0
    else:
        last = 1
    gather_wait(last)
    scat_start(last)
    scat_wait(1 - last)
    scat_wait(last)
    plsc.subcore_barrier()

    for t in range(NPT // WCH):
        r0 = s * NPT + t * WCH
        pltpu.sync_copy(accum.at[pl.ds(r0, WCH), :], rb0.at[pl.ds(0, WCH), :])
        pltpu.sync_copy(scale.at[pl.ds(r0, WCH), :],
                        rb0.at[pl.ds(WCH, WCH), :])
        if mode > 0:
            pltpu.sync_copy(extra.at[c].at[pl.ds(r0, WCH), :],
                            rb0.at[pl.ds(2 * WCH, WCH), :])

        def wrow(j, _):
            v = rb0[j] * rb0[WCH + j]
            if mode == 1:
                v = v + rb0[2 * WCH + j]
            elif mode == 2:
                v = v + rb0[2 * WCH + j]
                v = jnp.where(v > 0, v, 0.25 * v)
            rb0[j] = v
            return ()

        lax.fori_loop(0, WCH, wrow, ())
        pltpu.sync_copy(rb0.at[pl.ds(0, WCH), :], out.at[c].at[pl.ds(r0, WCH), :])


def _propagate(mode, tbl, eidx, scale, extra):
    body = functools.partial(_prop_body, mode)
    k = pl.kernel(
        body,
        out_type=jax.ShapeDtypeStruct((NC, N, D), _f32),
        mesh=_sc_mesh(),
        scratch_types=dict(
            accum=pltpu.VMEM_SHARED((N, D), _f32),
            eb0=pltpu.VMEM((2, CH), jnp.int32),
            eb1=pltpu.VMEM((2, CH), jnp.int32),
            rb0=pltpu.VMEM((CH, D), _f32),
            rb1=pltpu.VMEM((CH, D), _f32),
            gs0=pltpu.SemaphoreType.DMA,
            gs1=pltpu.SemaphoreType.DMA,
            ss0=pltpu.SemaphoreType.DMA,
            ss1=pltpu.SemaphoreType.DMA,
            is0=pltpu.SemaphoreType.DMA,
            is1=pltpu.SemaphoreType.DMA,
        ),
        compiler_params=pltpu.CompilerParams(use_tc_tiling_on_sc=False, needs_layout_passes=False),
        name=f"sc_prop_m{mode}",
    )
    if extra is None:
        extra = jnp.zeros((NC, 8, D), _f32)
    return k(tbl, eidx, scale, extra)


BM = 2048


def _pre_body(p_ref, cat_ref, noi_ref, u0_ref, d2e_ref, d1e_ref, sqe_ref):
    deg = jnp.sum(p_ref[...], axis=0)
    pos = deg > 0
    dinv = jnp.where(pos, lax.rsqrt(jnp.where(pos, deg, 1.0)), 0.0)
    dinv2 = jnp.where(pos, 1.0 / jnp.where(pos, deg, 1.0), 0.0)
    sqd = jnp.sqrt(deg)
    u0_ref[0] = dinv[:, None] * cat_ref[...]
    u0_ref[1] = dinv[:, None] * noi_ref[...]
    d2e_ref[...] = jnp.broadcast_to(dinv2[:, None], (BM, D))
    d1e_ref[...] = jnp.broadcast_to(dinv[:, None], (BM, D))
    sqe_ref[...] = jnp.broadcast_to(sqd[:, None], (BM, D))


def _tc_pre(partials, category, noise):
    grid = (N + BM - 1) // BM
    fb = jax.ShapeDtypeStruct((N, D), _f32)
    return pl.pallas_call(
        _pre_body,
        grid=(grid,),
        in_specs=[
            pl.BlockSpec((NW, BM), lambda i: (0, i)),
            pl.BlockSpec((BM, D), lambda i: (i, 0)),
            pl.BlockSpec((BM, D), lambda i: (i, 0)),
        ],
        out_specs=[
            pl.BlockSpec((NC, BM, D), lambda i: (0, i, 0)),
            pl.BlockSpec((BM, D), lambda i: (i, 0)),
            pl.BlockSpec((BM, D), lambda i: (i, 0)),
            pl.BlockSpec((BM, D), lambda i: (i, 0)),
        ],
        out_shape=[jax.ShapeDtypeStruct((NC, N, D), _f32), fb, fb, fb],
    )(partials, category, noise)


def _main_body(cat_ref, noi_ref, u1_ref, u2_ref, u3_ref, sqe_ref, d1e_ref,
               w1t_ref, b1_ref, w2r_ref, b2_ref,
               z0_ref, zt1_ref, zt2_ref, zt3_ref):
    s = sqe_ref[...]
    cat8 = jnp.concatenate(
        [cat_ref[...], noi_ref[...],
         s * u1_ref[0], s * u1_ref[1],
         s * u2_ref[0], s * u2_ref[1],
         s * u3_ref[0], s * u3_ref[1]], axis=1)
    y = jnp.dot(cat8, w1t_ref[...], preferred_element_type=_f32) + b1_ref[...]
    h = jnp.where(y > 0, y, 0.25 * y)
    w2r = w2r_ref[...]
    z0 = jnp.dot(h, w2r[:, :32], preferred_element_type=_f32) + b2_ref[...]
    hs = d1e_ref[:, :1] * h
    zs = jnp.dot(hs, w2r[:, 32:], preferred_element_type=_f32)
    z0_ref[0], z0_ref[1] = z0[:, :D], z0[:, D:]
    zt1_ref[0], zt1_ref[1] = zs[:, 0:D], zs[:, D:2 * D]
    zt2_ref[0], zt2_ref[1] = zs[:, 2 * D:3 * D], zs[:, 3 * D:4 * D]
    zt3_ref[0], zt3_ref[1] = zs[:, 4 * D:5 * D], zs[:, 5 * D:6 * D]


def _tc_main(category, noise, u1, u2, u3, sqe, d1e, W1, b1, W2, b2):
    grid = (N + BM - 1) // BM
    w1t = W1.T
    w2r = jnp.concatenate(
        [W2[:, 64 * j:64 * (j + 1)].T for j in range(4)], axis=1)
    fspec = pl.BlockSpec((BM, D), lambda i: (i, 0))
    uspec = pl.BlockSpec((NC, BM, D), lambda i: (0, i, 0))
    ut = jax.ShapeDtypeStruct((NC, N, D), _f32)
    return pl.pallas_call(
        _main_body,
        grid=(grid,),
        in_specs=[
            fspec, fspec, uspec, uspec, uspec, fspec, fspec,
            pl.BlockSpec((128, 64), lambda i: (0, 0)),
            pl.BlockSpec((1, 64), lambda i: (0, 0)),
            pl.BlockSpec((64, 128), lambda i: (0, 0)),
            pl.BlockSpec((1, 32), lambda i: (0, 0)),
        ],
        out_specs=[uspec, uspec, uspec, uspec],
        out_shape=[ut, ut, ut, ut],
    )(category, noise, u1, u2, u3, sqe, d1e,
      w1t, b1.reshape(1, 64), w2r, b2.reshape(1, 32))


def kernel(category, noise, edge_index, W1, b1, W2, b2):
    eidx = edge_index.astype(jnp.int32)
    dst = eidx[1]

    partials = _degree_partials(dst)
    u0, d2e, d1e, sqe = _tc_pre(partials, category, noise)

    u1 = _propagate(0, u0, eidx, d2e, None)
    u2 = _propagate(0, u1, eidx, d2e, None)
    u3 = _propagate(0, u2, eidx, d2e, None)

    z0, zt1, zt2, zt3 = _tc_main(category, noise, u1, u2, u3, sqe, d1e,
                                 W1, b1, W2, b2)

    w = _propagate(1, zt3, eidx, d2e, zt2)
    w = _propagate(1, w, eidx, d2e, zt1)
    o = _propagate(2, w, eidx, d1e, z0)

    return jnp.concatenate([o[0], o[1]], axis=1)
